# baseline (device time: 748208 ns/iter reference)
import jax
import jax.numpy as jnp
from jax import lax
from jax.experimental import pallas as pl
from jax.experimental.pallas import tpu as pltpu

M = 4096
N = 4096
KC = 1024

STREAMS = (
    (("x", "y", "z1", "z2"), 2560, 5),
    (("y", "z1", "z2", "x"), 1536, 3),
)
MAX_CHUNKS = max(s[2] for s in STREAMS)
HALF = (2048, 1024, 512, 256)


def kernel(dy, W):
    my_x = lax.axis_index("x")
    my_z = lax.axis_index("z")
    idx8 = my_x * 4 + my_z
    dy_c = lax.dynamic_slice(dy, (0, idx8 * KC), (M, KC))
    w_c = lax.dynamic_slice(W, (0, idx8 * KC), (N, KC))
    p = lax.dot_general(
        dy_c, w_c, (((1,), (1,)), ((), ())),
        preferred_element_type=jnp.float32,
    )

    n_flows = sum(s[2] for s in STREAMS)
    n_tids = 9 * n_flows

    def body(p_ref, o_ref, acc, recv, send_sems, recv_sems, ready, lsems):
        x = lax.axis_index("x")
        y = lax.axis_index("y")
        z = lax.axis_index("z")
        zb0 = z % 2
        zb1 = z // 2

        B = {"x": x, "y": y, "z1": zb0, "z2": zb1}
        P = {
            "x": (1 - x, y, z),
            "y": (x, 1 - y, z),
            "z1": (x, y, z + 1 - 2 * zb0),
            "z2": (x, y, z + 2 - 4 * zb1),
        }

        bar = pltpu.get_barrier_semaphore()
        for pid in P.values():
            pl.semaphore_signal(
                bar, inc=1, device_id=pid,
                device_id_type=pl.DeviceIdType.MESH,
            )
        pl.semaphore_wait(bar, 4)

        flows = []
        fi = 0
        col = 0
        for order, width, n_chunks in STREAMS:
            cw = width // n_chunks
            for c in range(n_chunks):
                f = {"order": order, "c0": col, "cw": cw, "fi": fi, "ci": c}
                fi += 1
                col += cw
                gb = 0
                gbs = [0]
                send_g = []
                for j, d in enumerate(order):
                    send_g.append(gb + (1 - B[d]) * HALF[j])
                    gb = gb + B[d] * HALF[j]
                    gbs.append(gb)
                f["gbs"] = gbs
                f["send_g"] = send_g
                flows.append(f)

        pend = {}

        def credit_and_start(tid, partner, src, dst):
            pl.semaphore_signal(
                ready.at[tid], inc=1, device_id=partner,
                device_id_type=pl.DeviceIdType.MESH,
            )
            pl.semaphore_wait(ready.at[tid], 1)
            rd = pltpu.make_async_remote_copy(
                src_ref=src, dst_ref=dst,
                send_sem=send_sems.at[tid], recv_sem=recv_sems.at[tid],
                device_id=partner, device_id_type=pl.DeviceIdType.MESH,
            )
            rd.start()
            return rd

        def issue(f, w):
            fi, c0, cw = f["fi"], f["c0"], f["cw"]
            gbs, order = f["gbs"], f["order"]
            cols = pl.ds(c0, cw)
            tid = 9 * fi + w
            if w in (0, 1):
                sub = w
                if w == 0:
                    f["cps"] = []
                    for s in range(2):
                        cp = pltpu.make_async_copy(
                            p_ref.at[pl.ds(gbs[1] + s * 1024, 1024), cols],
                            acc.at[pl.ds(s * 1024, 1024), cols],
                            lsems.at[3 * fi + s],
                        )
                        cp.start()
                        f["cps"].append(cp)
                rd = credit_and_start(
                    tid, P[order[0]],
                    p_ref.at[pl.ds(f["send_g"][0] + sub * 1024, 1024), cols],
                    recv.at[pl.ds(0, 1024), cols],
                )

                def fin(rd=rd, sub=sub, cp=f["cps"][sub], c0=c0, cw=cw):
                    rd.wait()
                    cp.wait()
                    r = pl.ds(sub * 1024, 1024)
                    acc[r, pl.ds(c0, cw)] = (
                        acc[r, pl.ds(c0, cw)] + recv[0:1024, c0:c0 + cw])
                pend[(fi, w)] = fin
            elif w in (2, 3, 4):
                j = w - 1
                n = HALF[j]
                cb_src = f["send_g"][j] - gbs[1]
                cb_dst = gbs[j + 1] - gbs[1]
                rd = credit_and_start(
                    tid, P[order[j]],
                    acc.at[pl.ds(cb_src, n), cols],
                    recv.at[pl.ds(0, n), cols],
                )

                def fin(rd=rd, n=n, cb_dst=cb_dst, c0=c0, cw=cw):
                    rd.wait()
                    r = pl.ds(cb_dst, n)
                    acc[r, pl.ds(c0, cw)] = (
                        acc[r, pl.ds(c0, cw)] + recv[0:n, c0:c0 + cw])
                pend[(fi, w)] = fin
            elif w == 5:
                cp = pltpu.make_async_copy(
                    acc.at[pl.ds(gbs[4] - gbs[1], 256), cols],
                    o_ref.at[pl.ds(gbs[4], 256), cols],
                    lsems.at[3 * fi + 2],
                )
                cp.start()
                rd = credit_and_start(
                    tid, P[order[3]],
                    acc.at[pl.ds(gbs[4] - gbs[1], 256), cols],
                    o_ref.at[pl.ds(gbs[4], 256), cols],
                )

                def fin(rd=rd, cp=cp):
                    rd.wait()
                    cp.wait()
                pend[(fi, w)] = fin
            else:
                k = w - 5
                jlev = 4 - k
                n = HALF[jlev - 1]
                rd = credit_and_start(
                    tid, P[order[jlev - 1]],
                    o_ref.at[pl.ds(gbs[jlev], n), cols],
                    o_ref.at[pl.ds(gbs[jlev], n), cols],
                )

                def fin(rd=rd):
                    rd.wait()
                pend[(fi, w)] = fin

        for t in range(9 + MAX_CHUNKS - 1):
            for f in flows:
                w = t - f["ci"]
                if 0 <= w <= 8:
                    if w > 0:
                        pend.pop((f["fi"], w - 1))()
                    issue(f, w)
        for f in flows:
            pend.pop((f["fi"], 8))()

    return pl.pallas_call(
        body,
        out_shape=jax.ShapeDtypeStruct((M, N), jnp.float32),
        in_specs=[pl.BlockSpec(memory_space=pl.ANY)],
        out_specs=pl.BlockSpec(memory_space=pl.ANY),
        scratch_shapes=[
            pltpu.VMEM((2048, N), jnp.float32),
            pltpu.VMEM((1024, N), jnp.float32),
            pltpu.SemaphoreType.DMA((n_tids,)),
            pltpu.SemaphoreType.DMA((n_tids,)),
            pltpu.SemaphoreType.REGULAR((n_tids,)),
            pltpu.SemaphoreType.DMA((3 * n_flows,)),
        ],
        compiler_params=pltpu.CompilerParams(
            collective_id=0,
            vmem_limit_bytes=56 * 1024 * 1024,
        ),
    )(p)


# device time: 592590 ns/iter; 1.2626x vs baseline; 1.2626x over previous
import jax
import jax.numpy as jnp
from jax import lax
from jax.experimental import pallas as pl
from jax.experimental.pallas import tpu as pltpu

M = 4096
N = 4096
K = 8192

STREAMS = (
    (("z2", "z1", "y", "x"), 2560, 2),
    (("z2", "z1", "x", "y"), 1536, 2),
)
MAX_CHUNKS = max(s[2] for s in STREAMS)
N_WAVES = 5


def kernel(dy, W):
    my_x = lax.axis_index("x")
    my_y = lax.axis_index("y")
    my_z = lax.axis_index("z")
    zb0 = my_z % 2
    zb1 = my_z // 2

    base_A = zb1 * 256 + zb0 * 512 + my_y * 1024 + my_x * 2048
    base_B = zb1 * 256 + zb0 * 512 + my_x * 1024 + my_y * 2048
    peer_A = base_A + (1 - 2 * my_y) * 1024
    peer_B = base_B + (1 - 2 * my_y) * 2048

    wA = STREAMS[0][1]
    w_sA = lax.slice(W, (0, 0), (wA, K))
    w_sB = lax.slice(W, (wA, 0), (N, K))
    dn = (((1,), (1,)), ((), ()))

    def part(base, w_s):
        d = lax.dynamic_slice(dy, (base, 0), (256, K))
        return lax.dot_general(d, w_s, dn, preferred_element_type=jnp.float32)

    pA_own = part(base_A, w_sA)
    pA_peer = part(peer_A, w_sA)
    pB_own = part(base_B, w_sB)
    pB_peer = part(peer_B, w_sB)

    n_flows = sum(s[2] for s in STREAMS)
    n_tids = N_WAVES * n_flows

    def body(pAo, pAp, pBo, pBp, o_ref, recv, stg,
             send_sems, recv_sems, lsems):
        x = lax.axis_index("x")
        y = lax.axis_index("y")
        z = lax.axis_index("z")
        zb0 = z % 2
        zb1 = z // 2

        B = {"x": x, "y": y, "z1": zb0, "z2": zb1}
        P = {
            "x": (1 - x, y, z),
            "y": (x, 1 - y, z),
            "z1": (x, y, z + 1 - 2 * zb0),
            "z2": (x, y, z + 2 - 4 * zb1),
        }

        bar = pltpu.get_barrier_semaphore()
        for pid in P.values():
            pl.semaphore_signal(
                bar, inc=1, device_id=pid,
                device_id_type=pl.DeviceIdType.MESH,
            )
        pl.semaphore_wait(bar, 4)

        flows = []
        fi = 0
        scol = 0
        stream_refs = ((pAo, pAp), (pBo, pBp))
        for si, (order, width, n_chunks) in enumerate(STREAMS):
            cw = width // n_chunks
            b = 0
            for j, d in enumerate(order):
                b = b + B[d] * (256 << j)
            bases = [b]
            for j, d in enumerate(order):
                bases.append(bases[-1] - B[d] * (256 << j))
            for c in range(n_chunks):
                flows.append({
                    "fi": fi, "ci": c, "order": order,
                    "own": stream_refs[si][0], "peer": stream_refs[si][1],
                    "c0l": c * cw, "c0g": scol + c * cw, "cw": cw,
                    "bases": bases,
                })
                fi += 1
            scol += width

        pend = {}

        def start_rdma(tid, partner, src, dst):
            rd = pltpu.make_async_remote_copy(
                src_ref=src, dst_ref=dst,
                send_sem=send_sems.at[tid], recv_sem=recv_sems.at[tid],
                device_id=partner, device_id_type=pl.DeviceIdType.MESH,
            )
            rd.start()
            return rd

        def issue(f, w):
            fi, cw, c0g = f["fi"], f["cw"], f["c0g"]
            colsg = pl.ds(c0g, cw)
            colsl = pl.ds(f["c0l"], cw)
            order, bases = f["order"], f["bases"]
            tid = N_WAVES * fi + w
            if w == 0:
                cp = pltpu.make_async_copy(
                    f["own"].at[pl.ds(0, 256), colsl],
                    stg.at[pl.ds(0, 256), colsg],
                    lsems.at[2 * fi],
                )
                cp.start()
                rd = start_rdma(
                    tid, P["y"],
                    f["peer"].at[pl.ds(0, 256), colsl],
                    recv.at[pl.ds(0, 256), colsg],
                )

                def fin(rd=rd, cp=cp, c0g=c0g, cw=cw):
                    rd.wait()
                    cp.wait()
                    r = pl.ds(0, 256)
                    cg = pl.ds(c0g, cw)
                    stg[r, cg] = stg[r, cg] + recv[0:256, c0g:c0g + cw]
                pend[(fi, w)] = fin
            elif w == 1:
                cp = pltpu.make_async_copy(
                    stg.at[pl.ds(0, 256), colsg],
                    o_ref.at[pl.ds(bases[0], 256), colsg],
                    lsems.at[2 * fi + 1],
                )
                cp.start()
                rd = start_rdma(
                    tid, P[order[0]],
                    stg.at[pl.ds(0, 256), colsg],
                    o_ref.at[pl.ds(bases[0], 256), colsg],
                )

                def fin(rd=rd, cp=cp):
                    rd.wait()
                    cp.wait()
                pend[(fi, w)] = fin
            else:
                n = 256 << (w - 1)
                rd = start_rdma(
                    tid, P[order[w - 1]],
                    o_ref.at[pl.ds(bases[w - 1], n), colsg],
                    o_ref.at[pl.ds(bases[w - 1], n), colsg],
                )

                def fin(rd=rd):
                    rd.wait()
                pend[(fi, w)] = fin

        for t in range(N_WAVES + MAX_CHUNKS - 1):
            for f in flows:
                w = t - f["ci"]
                if 0 <= w < N_WAVES:
                    if w > 0:
                        pend.pop((f["fi"], w - 1))()
                    issue(f, w)
        for f in flows:
            pend.pop((f["fi"], N_WAVES - 1))()

    return pl.pallas_call(
        body,
        out_shape=jax.ShapeDtypeStruct((M, N), jnp.float32),
        in_specs=[pl.BlockSpec(memory_space=pl.ANY)] * 4,
        out_specs=pl.BlockSpec(memory_space=pl.ANY),
        scratch_shapes=[
            pltpu.VMEM((256, N), jnp.float32),
            pltpu.VMEM((256, N), jnp.float32),
            pltpu.SemaphoreType.DMA((n_tids,)),
            pltpu.SemaphoreType.DMA((n_tids,)),
            pltpu.SemaphoreType.DMA((2 * n_flows,)),
        ],
        compiler_params=pltpu.CompilerParams(
            collective_id=0,
            vmem_limit_bytes=56 * 1024 * 1024,
        ),
    )(pA_own, pA_peer, pB_own, pB_peer)


# device time: 529239 ns/iter; 1.4137x vs baseline; 1.1197x over previous
import jax
import jax.numpy as jnp
from jax import lax
from jax.experimental import pallas as pl
from jax.experimental.pallas import tpu as pltpu

M = 4096
N = 4096
K = 8192

STREAMS = (
    (("z2", "z1", "y", "x"), 2560, 4),
    (("z2", "z1", "x", "y"), 1536, 4),
)
MAX_CHUNKS = max(s[2] for s in STREAMS)
N_WAVES = 5


def kernel(dy, W):
    my_x = lax.axis_index("x")
    my_y = lax.axis_index("y")
    my_z = lax.axis_index("z")
    zb0 = my_z % 2
    zb1 = my_z // 2

    base_A = zb1 * 256 + zb0 * 512 + my_y * 1024 + my_x * 2048
    base_B = zb1 * 256 + zb0 * 512 + my_x * 1024 + my_y * 2048
    peer_A = base_A + (1 - 2 * my_y) * 1024
    peer_B = base_B + (1 - 2 * my_y) * 2048

    wA = STREAMS[0][1]
    w_sA = lax.slice(W, (0, 0), (wA, K))
    w_sB = lax.slice(W, (wA, 0), (N, K))
    dn = (((1,), (1,)), ((), ()))

    def part(base, w_s):
        d = lax.dynamic_slice(dy, (base, 0), (256, K))
        return lax.dot_general(d, w_s, dn, preferred_element_type=jnp.float32)

    pA_own = part(base_A, w_sA)
    pA_peer = part(peer_A, w_sA)
    pB_own = part(base_B, w_sB)
    pB_peer = part(peer_B, w_sB)

    n_flows = sum(s[2] for s in STREAMS)
    n_tids = N_WAVES * n_flows

    def body(pAo, pAp, pBo, pBp, o_ref, recv, stg,
             send_sems, recv_sems, lsems):
        x = lax.axis_index("x")
        y = lax.axis_index("y")
        z = lax.axis_index("z")
        zb0 = z % 2
        zb1 = z // 2

        B = {"x": x, "y": y, "z1": zb0, "z2": zb1}
        P = {
            "x": (1 - x, y, z),
            "y": (x, 1 - y, z),
            "z1": (x, y, z + 1 - 2 * zb0),
            "z2": (x, y, z + 2 - 4 * zb1),
        }

        bar = pltpu.get_barrier_semaphore()
        for pid in P.values():
            pl.semaphore_signal(
                bar, inc=1, device_id=pid,
                device_id_type=pl.DeviceIdType.MESH,
            )
        pl.semaphore_wait(bar, 4)

        flows = []
        fi = 0
        scol = 0
        stream_refs = ((pAo, pAp), (pBo, pBp))
        for si, (order, width, n_chunks) in enumerate(STREAMS):
            cw = width // n_chunks
            b = 0
            for j, d in enumerate(order):
                b = b + B[d] * (256 << j)
            bases = [b]
            for j, d in enumerate(order):
                bases.append(bases[-1] - B[d] * (256 << j))
            for c in range(n_chunks):
                flows.append({
                    "fi": fi, "ci": c, "order": order,
                    "own": stream_refs[si][0], "peer": stream_refs[si][1],
                    "c0l": c * cw, "c0g": scol + c * cw, "cw": cw,
                    "bases": bases,
                })
                fi += 1
            scol += width

        pend = {}

        def start_rdma(tid, partner, src, dst):
            rd = pltpu.make_async_remote_copy(
                src_ref=src, dst_ref=dst,
                send_sem=send_sems.at[tid], recv_sem=recv_sems.at[tid],
                device_id=partner, device_id_type=pl.DeviceIdType.MESH,
            )
            rd.start()
            return rd

        def issue(f, w):
            fi, cw, c0g = f["fi"], f["cw"], f["c0g"]
            colsg = pl.ds(c0g, cw)
            colsl = pl.ds(f["c0l"], cw)
            order, bases = f["order"], f["bases"]
            tid = N_WAVES * fi + w
            if w == 0:
                cp = pltpu.make_async_copy(
                    f["own"].at[pl.ds(0, 256), colsl],
                    stg.at[pl.ds(0, 256), colsg],
                    lsems.at[2 * fi],
                )
                cp.start()
                rd = start_rdma(
                    tid, P["y"],
                    f["peer"].at[pl.ds(0, 256), colsl],
                    recv.at[pl.ds(0, 256), colsg],
                )

                def fin(rd=rd, cp=cp, c0g=c0g, cw=cw):
                    rd.wait()
                    cp.wait()
                    r = pl.ds(0, 256)
                    cg = pl.ds(c0g, cw)
                    stg[r, cg] = stg[r, cg] + recv[0:256, c0g:c0g + cw]
                pend[(fi, w)] = fin
            elif w == 1:
                cp = pltpu.make_async_copy(
                    stg.at[pl.ds(0, 256), colsg],
                    o_ref.at[pl.ds(bases[0], 256), colsg],
                    lsems.at[2 * fi + 1],
                )
                cp.start()
                rd = start_rdma(
                    tid, P[order[0]],
                    stg.at[pl.ds(0, 256), colsg],
                    o_ref.at[pl.ds(bases[0], 256), colsg],
                )

                def fin(rd=rd, cp=cp):
                    rd.wait()
                    cp.wait()
                pend[(fi, w)] = fin
            else:
                n = 256 << (w - 1)
                rd = start_rdma(
                    tid, P[order[w - 1]],
                    o_ref.at[pl.ds(bases[w - 1], n), colsg],
                    o_ref.at[pl.ds(bases[w - 1], n), colsg],
                )

                def fin(rd=rd):
                    rd.wait()
                pend[(fi, w)] = fin

        for t in range(N_WAVES + MAX_CHUNKS - 1):
            for f in flows:
                w = t - f["ci"]
                if 0 <= w < N_WAVES:
                    if w > 0:
                        pend.pop((f["fi"], w - 1))()
                    issue(f, w)
        for f in flows:
            pend.pop((f["fi"], N_WAVES - 1))()

    return pl.pallas_call(
        body,
        out_shape=jax.ShapeDtypeStruct((M, N), jnp.float32),
        in_specs=[pl.BlockSpec(memory_space=pl.ANY)] * 4,
        out_specs=pl.BlockSpec(memory_space=pl.ANY),
        scratch_shapes=[
            pltpu.VMEM((256, N), jnp.float32),
            pltpu.VMEM((256, N), jnp.float32),
            pltpu.SemaphoreType.DMA((n_tids,)),
            pltpu.SemaphoreType.DMA((n_tids,)),
            pltpu.SemaphoreType.DMA((2 * n_flows,)),
        ],
        compiler_params=pltpu.CompilerParams(
            collective_id=0,
            vmem_limit_bytes=56 * 1024 * 1024,
        ),
    )(pA_own, pA_peer, pB_own, pB_peer)


# device time: 494334 ns/iter; 1.5136x vs baseline; 1.0706x over previous
import jax
import jax.numpy as jnp
from jax import lax
from jax.experimental import pallas as pl
from jax.experimental.pallas import tpu as pltpu

M = 4096
N = 4096
K = 8192

STREAMS = (
    (("z2", "z1", "y", "x"), 2560, 5),
    (("z2", "z1", "x", "y"), 1536, 3),
)
MAX_CHUNKS = max(s[2] for s in STREAMS)
N_WAVES = 5


def kernel(dy, W):
    my_x = lax.axis_index("x")
    my_y = lax.axis_index("y")
    my_z = lax.axis_index("z")
    zb0 = my_z % 2
    zb1 = my_z // 2

    base_A = zb1 * 256 + zb0 * 512 + my_y * 1024 + my_x * 2048
    base_B = zb1 * 256 + zb0 * 512 + my_x * 1024 + my_y * 2048
    peer_A = base_A + (1 - 2 * my_y) * 1024
    peer_B = base_B + (1 - 2 * my_y) * 2048

    wA = STREAMS[0][1]
    w_sA = lax.slice(W, (0, 0), (wA, K))
    w_sB = lax.slice(W, (wA, 0), (N, K))
    dn = (((1,), (1,)), ((), ()))

    def part(base, peer, w_s):
        d = jnp.concatenate([
            lax.dynamic_slice(dy, (base, 0), (256, K)),
            lax.dynamic_slice(dy, (peer, 0), (256, K)),
        ])
        return lax.dot_general(d, w_s, dn, preferred_element_type=jnp.float32)

    pA = part(base_A, peer_A, w_sA)
    pB = part(base_B, peer_B, w_sB)

    n_flows = sum(s[2] for s in STREAMS)
    n_tids = N_WAVES * n_flows

    def body(pA_ref, pB_ref, o_ref, recv, stg,
             send_sems, recv_sems, lsems):
        x = lax.axis_index("x")
        y = lax.axis_index("y")
        z = lax.axis_index("z")
        zb0 = z % 2
        zb1 = z // 2

        B = {"x": x, "y": y, "z1": zb0, "z2": zb1}
        P = {
            "x": (1 - x, y, z),
            "y": (x, 1 - y, z),
            "z1": (x, y, z + 1 - 2 * zb0),
            "z2": (x, y, z + 2 - 4 * zb1),
        }

        bar = pltpu.get_barrier_semaphore()
        for pid in P.values():
            pl.semaphore_signal(
                bar, inc=1, device_id=pid,
                device_id_type=pl.DeviceIdType.MESH,
            )
        pl.semaphore_wait(bar, 4)

        flows = []
        fi = 0
        scol = 0
        stream_refs = (pA_ref, pB_ref)
        for si, (order, width, n_chunks) in enumerate(STREAMS):
            cw = width // n_chunks
            b = 0
            for j, d in enumerate(order):
                b = b + B[d] * (256 << j)
            bases = [b]
            for j, d in enumerate(order):
                bases.append(bases[-1] - B[d] * (256 << j))
            for c in range(n_chunks):
                flows.append({
                    "fi": fi, "ci": c, "order": order,
                    "p": stream_refs[si],
                    "c0l": c * cw, "c0g": scol + c * cw, "cw": cw,
                    "bases": bases,
                })
                fi += 1
            scol += width

        pend = {}

        def start_rdma(tid, partner, src, dst):
            rd = pltpu.make_async_remote_copy(
                src_ref=src, dst_ref=dst,
                send_sem=send_sems.at[tid], recv_sem=recv_sems.at[tid],
                device_id=partner, device_id_type=pl.DeviceIdType.MESH,
            )
            rd.start()
            return rd

        def issue(f, w):
            fi, cw, c0g = f["fi"], f["cw"], f["c0g"]
            colsg = pl.ds(c0g, cw)
            colsl = pl.ds(f["c0l"], cw)
            order, bases = f["order"], f["bases"]
            tid = N_WAVES * fi + w
            if w == 0:
                cp = pltpu.make_async_copy(
                    f["p"].at[pl.ds(0, 256), colsl],
                    stg.at[pl.ds(0, 256), colsg],
                    lsems.at[2 * fi],
                )
                cp.start()
                rd = start_rdma(
                    tid, P["y"],
                    f["p"].at[pl.ds(256, 256), colsl],
                    recv.at[pl.ds(0, 256), colsg],
                )

                def fin(rd=rd, cp=cp, c0g=c0g, cw=cw):
                    rd.wait()
                    cp.wait()
                    r = pl.ds(0, 256)
                    cg = pl.ds(c0g, cw)
                    stg[r, cg] = stg[r, cg] + recv[0:256, c0g:c0g + cw]
                pend[(fi, w)] = fin
            elif w == 1:
                cp = pltpu.make_async_copy(
                    stg.at[pl.ds(0, 256), colsg],
                    o_ref.at[pl.ds(bases[0], 256), colsg],
                    lsems.at[2 * fi + 1],
                )
                cp.start()
                rd = start_rdma(
                    tid, P[order[0]],
                    stg.at[pl.ds(0, 256), colsg],
                    o_ref.at[pl.ds(bases[0], 256), colsg],
                )

                def fin(rd=rd, cp=cp):
                    rd.wait()
                    cp.wait()
                pend[(fi, w)] = fin
            else:
                n = 256 << (w - 1)
                rd = start_rdma(
                    tid, P[order[w - 1]],
                    o_ref.at[pl.ds(bases[w - 1], n), colsg],
                    o_ref.at[pl.ds(bases[w - 1], n), colsg],
                )

                def fin(rd=rd):
                    rd.wait()
                pend[(fi, w)] = fin

        for t in range(N_WAVES + MAX_CHUNKS - 1):
            for f in flows:
                w = t - f["ci"]
                if 0 <= w < N_WAVES:
                    if w > 0:
                        pend.pop((f["fi"], w - 1))()
                    issue(f, w)
        for f in flows:
            pend.pop((f["fi"], N_WAVES - 1))()

    return pl.pallas_call(
        body,
        out_shape=jax.ShapeDtypeStruct((M, N), jnp.float32),
        in_specs=[pl.BlockSpec(memory_space=pl.ANY)] * 2,
        out_specs=pl.BlockSpec(memory_space=pl.ANY),
        scratch_shapes=[
            pltpu.VMEM((256, N), jnp.float32),
            pltpu.VMEM((256, N), jnp.float32),
            pltpu.SemaphoreType.DMA((n_tids,)),
            pltpu.SemaphoreType.DMA((n_tids,)),
            pltpu.SemaphoreType.DMA((2 * n_flows,)),
        ],
        compiler_params=pltpu.CompilerParams(
            collective_id=0,
            vmem_limit_bytes=56 * 1024 * 1024,
        ),
    )(pA, pB)


# device time: 481198 ns/iter; 1.5549x vs baseline; 1.0273x over previous
import jax
import jax.numpy as jnp
from jax import lax
from jax.experimental import pallas as pl
from jax.experimental.pallas import tpu as pltpu

M = 4096
N = 4096
K = 8192

STREAMS = (
    (("z2", "z1", "y", "x"), 2560, 10),
    (("z2", "z1", "x", "y"), 1536, 6),
)
MAX_CHUNKS = max(s[2] for s in STREAMS)
N_WAVES = 5


def kernel(dy, W):
    my_x = lax.axis_index("x")
    my_y = lax.axis_index("y")
    my_z = lax.axis_index("z")
    zb0 = my_z % 2
    zb1 = my_z // 2

    base_A = zb1 * 256 + zb0 * 512 + my_y * 1024 + my_x * 2048
    base_B = zb1 * 256 + zb0 * 512 + my_x * 1024 + my_y * 2048
    peer_A = base_A + (1 - 2 * my_y) * 1024
    peer_B = base_B + (1 - 2 * my_y) * 2048

    wA = STREAMS[0][1]
    w_sA = lax.slice(W, (0, 0), (wA, K))
    w_sB = lax.slice(W, (wA, 0), (N, K))
    dn = (((1,), (1,)), ((), ()))

    def part(base, peer, w_s):
        d = jnp.concatenate([
            lax.dynamic_slice(dy, (base, 0), (256, K)),
            lax.dynamic_slice(dy, (peer, 0), (256, K)),
        ])
        return lax.dot_general(d, w_s, dn, preferred_element_type=jnp.float32)

    pA = part(base_A, peer_A, w_sA)
    pB = part(base_B, peer_B, w_sB)

    n_flows = sum(s[2] for s in STREAMS)
    n_tids = N_WAVES * n_flows

    def body(pA_ref, pB_ref, o_ref, recv, stg,
             send_sems, recv_sems, lsems):
        x = lax.axis_index("x")
        y = lax.axis_index("y")
        z = lax.axis_index("z")
        zb0 = z % 2
        zb1 = z // 2

        B = {"x": x, "y": y, "z1": zb0, "z2": zb1}
        P = {
            "x": (1 - x, y, z),
            "y": (x, 1 - y, z),
            "z1": (x, y, z + 1 - 2 * zb0),
            "z2": (x, y, z + 2 - 4 * zb1),
        }

        bar = pltpu.get_barrier_semaphore()
        for pid in P.values():
            pl.semaphore_signal(
                bar, inc=1, device_id=pid,
                device_id_type=pl.DeviceIdType.MESH,
            )
        pl.semaphore_wait(bar, 4)

        flows = []
        fi = 0
        scol = 0
        stream_refs = (pA_ref, pB_ref)
        for si, (order, width, n_chunks) in enumerate(STREAMS):
            cw = width // n_chunks
            b = 0
            for j, d in enumerate(order):
                b = b + B[d] * (256 << j)
            bases = [b]
            for j, d in enumerate(order):
                bases.append(bases[-1] - B[d] * (256 << j))
            for c in range(n_chunks):
                flows.append({
                    "fi": fi, "ci": c, "order": order,
                    "p": stream_refs[si],
                    "c0l": c * cw, "c0g": scol + c * cw, "cw": cw,
                    "bases": bases,
                })
                fi += 1
            scol += width

        pend = {}

        def start_rdma(tid, partner, src, dst):
            rd = pltpu.make_async_remote_copy(
                src_ref=src, dst_ref=dst,
                send_sem=send_sems.at[tid], recv_sem=recv_sems.at[tid],
                device_id=partner, device_id_type=pl.DeviceIdType.MESH,
            )
            rd.start()
            return rd

        def issue(f, w):
            fi, cw, c0g = f["fi"], f["cw"], f["c0g"]
            colsg = pl.ds(c0g, cw)
            colsl = pl.ds(f["c0l"], cw)
            order, bases = f["order"], f["bases"]
            tid = N_WAVES * fi + w
            if w == 0:
                cp = pltpu.make_async_copy(
                    f["p"].at[pl.ds(0, 256), colsl],
                    stg.at[pl.ds(0, 256), colsg],
                    lsems.at[2 * fi],
                )
                cp.start()
                rd = start_rdma(
                    tid, P["y"],
                    f["p"].at[pl.ds(256, 256), colsl],
                    recv.at[pl.ds(0, 256), colsg],
                )

                def fin(rd=rd, cp=cp, c0g=c0g, cw=cw):
                    rd.wait()
                    cp.wait()
                    r = pl.ds(0, 256)
                    cg = pl.ds(c0g, cw)
                    stg[r, cg] = stg[r, cg] + recv[0:256, c0g:c0g + cw]
                pend[(fi, w)] = fin
            elif w == 1:
                cp = pltpu.make_async_copy(
                    stg.at[pl.ds(0, 256), colsg],
                    o_ref.at[pl.ds(bases[0], 256), colsg],
                    lsems.at[2 * fi + 1],
                )
                cp.start()
                rd = start_rdma(
                    tid, P[order[0]],
                    stg.at[pl.ds(0, 256), colsg],
                    o_ref.at[pl.ds(bases[0], 256), colsg],
                )

                def fin(rd=rd, cp=cp):
                    rd.wait()
                    cp.wait()
                pend[(fi, w)] = fin
            else:
                n = 256 << (w - 1)
                rd = start_rdma(
                    tid, P[order[w - 1]],
                    o_ref.at[pl.ds(bases[w - 1], n), colsg],
                    o_ref.at[pl.ds(bases[w - 1], n), colsg],
                )

                def fin(rd=rd):
                    rd.wait()
                pend[(fi, w)] = fin

        for t in range(N_WAVES + MAX_CHUNKS - 1):
            for f in flows:
                w = t - f["ci"]
                if 0 <= w < N_WAVES:
                    if w > 0:
                        pend.pop((f["fi"], w - 1))()
                    issue(f, w)
        for f in flows:
            pend.pop((f["fi"], N_WAVES - 1))()

    return pl.pallas_call(
        body,
        out_shape=jax.ShapeDtypeStruct((M, N), jnp.float32),
        in_specs=[pl.BlockSpec(memory_space=pl.ANY)] * 2,
        out_specs=pl.BlockSpec(memory_space=pl.ANY),
        scratch_shapes=[
            pltpu.VMEM((256, N), jnp.float32),
            pltpu.VMEM((256, N), jnp.float32),
            pltpu.SemaphoreType.DMA((n_tids,)),
            pltpu.SemaphoreType.DMA((n_tids,)),
            pltpu.SemaphoreType.DMA((2 * n_flows,)),
        ],
        compiler_params=pltpu.CompilerParams(
            collective_id=0,
            vmem_limit_bytes=56 * 1024 * 1024,
        ),
    )(pA, pB)


# device time: 472873 ns/iter; 1.5823x vs baseline; 1.0176x over previous
import jax
import jax.numpy as jnp
from jax import lax
from jax.experimental import pallas as pl
from jax.experimental.pallas import tpu as pltpu

M = 4096
N = 4096
K = 8192

STREAMS = (
    (("z2", "z1", "y", "x"), 2560, 20),
    (("z2", "z1", "x", "y"), 1536, 12),
)
MAX_CHUNKS = max(s[2] for s in STREAMS)
N_WAVES = 5


def kernel(dy, W):
    my_x = lax.axis_index("x")
    my_y = lax.axis_index("y")
    my_z = lax.axis_index("z")
    zb0 = my_z % 2
    zb1 = my_z // 2

    base_A = zb1 * 256 + zb0 * 512 + my_y * 1024 + my_x * 2048
    base_B = zb1 * 256 + zb0 * 512 + my_x * 1024 + my_y * 2048
    peer_A = base_A + (1 - 2 * my_y) * 1024
    peer_B = base_B + (1 - 2 * my_y) * 2048

    wA = STREAMS[0][1]
    w_sA = lax.slice(W, (0, 0), (wA, K))
    w_sB = lax.slice(W, (wA, 0), (N, K))
    dn = (((1,), (1,)), ((), ()))

    def part(base, peer, w_s):
        d = jnp.concatenate([
            lax.dynamic_slice(dy, (base, 0), (256, K)),
            lax.dynamic_slice(dy, (peer, 0), (256, K)),
        ])
        return lax.dot_general(d, w_s, dn, preferred_element_type=jnp.float32)

    pA = part(base_A, peer_A, w_sA)
    pB = part(base_B, peer_B, w_sB)

    n_flows = sum(s[2] for s in STREAMS)
    n_tids = N_WAVES * n_flows

    def body(pA_ref, pB_ref, o_ref, recv, stg,
             send_sems, recv_sems, lsems):
        x = lax.axis_index("x")
        y = lax.axis_index("y")
        z = lax.axis_index("z")
        zb0 = z % 2
        zb1 = z // 2

        B = {"x": x, "y": y, "z1": zb0, "z2": zb1}
        P = {
            "x": (1 - x, y, z),
            "y": (x, 1 - y, z),
            "z1": (x, y, z + 1 - 2 * zb0),
            "z2": (x, y, z + 2 - 4 * zb1),
        }

        bar = pltpu.get_barrier_semaphore()
        for pid in P.values():
            pl.semaphore_signal(
                bar, inc=1, device_id=pid,
                device_id_type=pl.DeviceIdType.MESH,
            )
        pl.semaphore_wait(bar, 4)

        flows = []
        fi = 0
        scol = 0
        stream_refs = (pA_ref, pB_ref)
        for si, (order, width, n_chunks) in enumerate(STREAMS):
            cw = width // n_chunks
            b = 0
            for j, d in enumerate(order):
                b = b + B[d] * (256 << j)
            bases = [b]
            for j, d in enumerate(order):
                bases.append(bases[-1] - B[d] * (256 << j))
            for c in range(n_chunks):
                flows.append({
                    "fi": fi, "ci": c, "order": order,
                    "p": stream_refs[si],
                    "c0l": c * cw, "c0g": scol + c * cw, "cw": cw,
                    "bases": bases,
                })
                fi += 1
            scol += width

        pend = {}

        def start_rdma(tid, partner, src, dst):
            rd = pltpu.make_async_remote_copy(
                src_ref=src, dst_ref=dst,
                send_sem=send_sems.at[tid], recv_sem=recv_sems.at[tid],
                device_id=partner, device_id_type=pl.DeviceIdType.MESH,
            )
            rd.start()
            return rd

        def issue(f, w):
            fi, cw, c0g = f["fi"], f["cw"], f["c0g"]
            colsg = pl.ds(c0g, cw)
            colsl = pl.ds(f["c0l"], cw)
            order, bases = f["order"], f["bases"]
            tid = N_WAVES * fi + w
            if w == 0:
                cp = pltpu.make_async_copy(
                    f["p"].at[pl.ds(0, 256), colsl],
                    stg.at[pl.ds(0, 256), colsg],
                    lsems.at[2 * fi],
                )
                cp.start()
                rd = start_rdma(
                    tid, P["y"],
                    f["p"].at[pl.ds(256, 256), colsl],
                    recv.at[pl.ds(0, 256), colsg],
                )

                def fin(rd=rd, cp=cp, c0g=c0g, cw=cw):
                    rd.wait()
                    cp.wait()
                    r = pl.ds(0, 256)
                    cg = pl.ds(c0g, cw)
                    stg[r, cg] = stg[r, cg] + recv[0:256, c0g:c0g + cw]
                pend[(fi, w)] = fin
            elif w == 1:
                cp = pltpu.make_async_copy(
                    stg.at[pl.ds(0, 256), colsg],
                    o_ref.at[pl.ds(bases[0], 256), colsg],
                    lsems.at[2 * fi + 1],
                )
                cp.start()
                rd = start_rdma(
                    tid, P[order[0]],
                    stg.at[pl.ds(0, 256), colsg],
                    o_ref.at[pl.ds(bases[0], 256), colsg],
                )

                def fin(rd=rd, cp=cp):
                    rd.wait()
                    cp.wait()
                pend[(fi, w)] = fin
            else:
                n = 256 << (w - 1)
                rd = start_rdma(
                    tid, P[order[w - 1]],
                    o_ref.at[pl.ds(bases[w - 1], n), colsg],
                    o_ref.at[pl.ds(bases[w - 1], n), colsg],
                )

                def fin(rd=rd):
                    rd.wait()
                pend[(fi, w)] = fin

        for t in range(N_WAVES + MAX_CHUNKS - 1):
            for f in flows:
                w = t - f["ci"]
                if 0 <= w < N_WAVES:
                    if w > 0:
                        pend.pop((f["fi"], w - 1))()
                    issue(f, w)
        for f in flows:
            pend.pop((f["fi"], N_WAVES - 1))()

    return pl.pallas_call(
        body,
        out_shape=jax.ShapeDtypeStruct((M, N), jnp.float32),
        in_specs=[pl.BlockSpec(memory_space=pl.ANY)] * 2,
        out_specs=pl.BlockSpec(memory_space=pl.ANY),
        scratch_shapes=[
            pltpu.VMEM((256, N), jnp.float32),
            pltpu.VMEM((256, N), jnp.float32),
            pltpu.SemaphoreType.DMA((n_tids,)),
            pltpu.SemaphoreType.DMA((n_tids,)),
            pltpu.SemaphoreType.DMA((2 * n_flows,)),
        ],
        compiler_params=pltpu.CompilerParams(
            collective_id=0,
            vmem_limit_bytes=56 * 1024 * 1024,
        ),
    )(pA, pB)


# device time: 472538 ns/iter; 1.5834x vs baseline; 1.0007x over previous
import jax
import jax.numpy as jnp
from jax import lax
from jax.experimental import pallas as pl
from jax.experimental.pallas import tpu as pltpu

M = 4096
N = 4096
K = 8192

STREAMS = (
    (("z2", "z1", "y", "x"), 2560, 20),
    (("z1", "z2", "x", "y"), 1536, 12),
)
MAX_CHUNKS = max(s[2] for s in STREAMS)
N_WAVES = 5


def kernel(dy, W):
    my_x = lax.axis_index("x")
    my_y = lax.axis_index("y")
    my_z = lax.axis_index("z")
    zb0 = my_z % 2
    zb1 = my_z // 2

    base_A = zb1 * 256 + zb0 * 512 + my_y * 1024 + my_x * 2048
    base_B = zb1 * 256 + zb0 * 512 + my_x * 1024 + my_y * 2048
    peer_A = base_A + (1 - 2 * my_y) * 1024
    peer_B = base_B + (1 - 2 * my_y) * 2048

    wA = STREAMS[0][1]
    w_sA = lax.slice(W, (0, 0), (wA, K))
    w_sB = lax.slice(W, (wA, 0), (N, K))
    dn = (((1,), (1,)), ((), ()))

    def part(base, peer, w_s):
        d = jnp.concatenate([
            lax.dynamic_slice(dy, (base, 0), (256, K)),
            lax.dynamic_slice(dy, (peer, 0), (256, K)),
        ])
        return lax.dot_general(d, w_s, dn, preferred_element_type=jnp.float32)

    pA = part(base_A, peer_A, w_sA)
    pB = part(base_B, peer_B, w_sB)

    n_flows = sum(s[2] for s in STREAMS)
    n_tids = N_WAVES * n_flows

    def body(pA_ref, pB_ref, o_ref, recv, stg,
             send_sems, recv_sems, lsems):
        x = lax.axis_index("x")
        y = lax.axis_index("y")
        z = lax.axis_index("z")
        zb0 = z % 2
        zb1 = z // 2

        B = {"x": x, "y": y, "z1": zb0, "z2": zb1}
        P = {
            "x": (1 - x, y, z),
            "y": (x, 1 - y, z),
            "z1": (x, y, z + 1 - 2 * zb0),
            "z2": (x, y, z + 2 - 4 * zb1),
        }

        bar = pltpu.get_barrier_semaphore()
        for pid in P.values():
            pl.semaphore_signal(
                bar, inc=1, device_id=pid,
                device_id_type=pl.DeviceIdType.MESH,
            )
        pl.semaphore_wait(bar, 4)

        flows = []
        fi = 0
        scol = 0
        stream_refs = (pA_ref, pB_ref)
        for si, (order, width, n_chunks) in enumerate(STREAMS):
            cw = width // n_chunks
            b = 0
            for j, d in enumerate(order):
                b = b + B[d] * (256 << j)
            bases = [b]
            for j, d in enumerate(order):
                bases.append(bases[-1] - B[d] * (256 << j))
            for c in range(n_chunks):
                flows.append({
                    "fi": fi, "ci": c, "order": order,
                    "p": stream_refs[si],
                    "c0l": c * cw, "c0g": scol + c * cw, "cw": cw,
                    "bases": bases,
                })
                fi += 1
            scol += width

        pend = {}

        def start_rdma(tid, partner, src, dst):
            rd = pltpu.make_async_remote_copy(
                src_ref=src, dst_ref=dst,
                send_sem=send_sems.at[tid], recv_sem=recv_sems.at[tid],
                device_id=partner, device_id_type=pl.DeviceIdType.MESH,
            )
            rd.start()
            return rd

        def issue(f, w):
            fi, cw, c0g = f["fi"], f["cw"], f["c0g"]
            colsg = pl.ds(c0g, cw)
            colsl = pl.ds(f["c0l"], cw)
            order, bases = f["order"], f["bases"]
            tid = N_WAVES * fi + w
            if w == 0:
                cp = pltpu.make_async_copy(
                    f["p"].at[pl.ds(0, 256), colsl],
                    stg.at[pl.ds(0, 256), colsg],
                    lsems.at[2 * fi],
                )
                cp.start()
                rd = start_rdma(
                    tid, P["y"],
                    f["p"].at[pl.ds(256, 256), colsl],
                    recv.at[pl.ds(0, 256), colsg],
                )

                def fin(rd=rd, cp=cp, c0g=c0g, cw=cw):
                    rd.wait()
                    cp.wait()
                    r = pl.ds(0, 256)
                    cg = pl.ds(c0g, cw)
                    stg[r, cg] = stg[r, cg] + recv[0:256, c0g:c0g + cw]
                pend[(fi, w)] = fin
            elif w == 1:
                cp = pltpu.make_async_copy(
                    stg.at[pl.ds(0, 256), colsg],
                    o_ref.at[pl.ds(bases[0], 256), colsg],
                    lsems.at[2 * fi + 1],
                )
                cp.start()
                rd = start_rdma(
                    tid, P[order[0]],
                    stg.at[pl.ds(0, 256), colsg],
                    o_ref.at[pl.ds(bases[0], 256), colsg],
                )

                def fin(rd=rd, cp=cp):
                    rd.wait()
                    cp.wait()
                pend[(fi, w)] = fin
            else:
                n = 256 << (w - 1)
                rd = start_rdma(
                    tid, P[order[w - 1]],
                    o_ref.at[pl.ds(bases[w - 1], n), colsg],
                    o_ref.at[pl.ds(bases[w - 1], n), colsg],
                )

                def fin(rd=rd):
                    rd.wait()
                pend[(fi, w)] = fin

        for t in range(N_WAVES + MAX_CHUNKS - 1):
            for f in flows:
                w = t - f["ci"]
                if 0 <= w < N_WAVES:
                    if w > 0:
                        pend.pop((f["fi"], w - 1))()
                    issue(f, w)
        for f in flows:
            pend.pop((f["fi"], N_WAVES - 1))()

    return pl.pallas_call(
        body,
        out_shape=jax.ShapeDtypeStruct((M, N), jnp.float32),
        in_specs=[pl.BlockSpec(memory_space=pl.ANY)] * 2,
        out_specs=pl.BlockSpec(memory_space=pl.ANY),
        scratch_shapes=[
            pltpu.VMEM((256, N), jnp.float32),
            pltpu.VMEM((256, N), jnp.float32),
            pltpu.SemaphoreType.DMA((n_tids,)),
            pltpu.SemaphoreType.DMA((n_tids,)),
            pltpu.SemaphoreType.DMA((2 * n_flows,)),
        ],
        compiler_params=pltpu.CompilerParams(
            collective_id=0,
            vmem_limit_bytes=56 * 1024 * 1024,
        ),
    )(pA, pB)
